# Initial kernel scaffold; baseline (speedup 1.0000x reference)
import functools

import jax
import jax.numpy as jnp
from jax import lax
from jax.experimental import pallas as pl
from jax.experimental.pallas import tpu as pltpu
from jax.experimental.pallas import tpu_sc as plsc

NC = 2
NS = 16
CHUNK = 128


def _tile_1d_ranges(n):
  per = (n // NS) // 8 * 8
  ranges = [(w * per, per) for w in range(NS)]
  tail_off = per * NS
  tail = n - tail_off
  extra = (tail_off, tail) if tail else None
  return ranges, extra


def _deg_kernel(num_inc, num_he, num_nodes):
  mesh = plsc.VectorSubcoreMesh(core_axis_name="c", subcore_axis_name="s")
  n_chunks = num_inc // CHUNK
  per_core = n_chunks // NC
  full = per_core // NS
  tail_tiles = per_core % NS

  @functools.partial(
      pl.kernel,
      out_type=[
          jax.ShapeDtypeStruct((NC, num_nodes), jnp.float32),
          jax.ShapeDtypeStruct((NC, num_he), jnp.float32),
      ],
      mesh=mesh,
      scratch_types=[
          pltpu.VMEM((CHUNK,), jnp.int32),
          pltpu.VMEM((CHUNK,), jnp.int32),
          pltpu.VMEM((CHUNK,), jnp.float32),
          pltpu.VMEM((CHUNK,), jnp.float32),
          pltpu.VMEM_SHARED((num_nodes,), jnp.float32),
          pltpu.VMEM_SHARED((num_he,), jnp.float32),
      ],
  )
  def deg(nidx_hbm, hidx_hbm, w_hbm, z1_hbm, dv_hbm, de_hbm,
          ni, hi, wv, ones, dv_acc, de_acc):
    c = lax.axis_index("c")
    s = lax.axis_index("s")

    @pl.loop(0, CHUNK, step=16)
    def _(i):
      ones[pl.ds(i, 16)] = jnp.full((16,), 1.0, jnp.float32)

    ranges, extra = _tile_1d_ranges(num_nodes)
    for w, (off, sz) in enumerate(ranges):
      @pl.when(s == w)
      def _():
        pltpu.sync_copy(z1_hbm.at[pl.ds(off, sz)], dv_acc.at[pl.ds(off, sz)])
        pltpu.sync_copy(z1_hbm.at[pl.ds(off, sz)], de_acc.at[pl.ds(off, sz)])
    if extra is not None:
      off, sz = extra
      @pl.when(s == 0)
      def _():
        pltpu.sync_copy(z1_hbm.at[pl.ds(off, sz)], dv_acc.at[pl.ds(off, sz)])
        pltpu.sync_copy(z1_hbm.at[pl.ds(off, sz)], de_acc.at[pl.ds(off, sz)])
    plsc.subcore_barrier()

    def do_chunk(chunk_id):
      base = chunk_id * CHUNK
      pltpu.sync_copy(nidx_hbm.at[pl.ds(base, CHUNK)], ni)
      pltpu.sync_copy(hidx_hbm.at[pl.ds(base, CHUNK)], hi)
      pltpu.sync_copy(w_hbm.at[hi], wv)
      pltpu.sync_copy(ones, de_acc.at[hi], add=True)
      pltpu.sync_copy(wv, dv_acc.at[ni], add=True)

    @pl.loop(0, full)
    def _(j):
      do_chunk(c * per_core + s + NS * j)

    if tail_tiles:
      @pl.when(s < tail_tiles)
      def _():
        do_chunk(c * per_core + NS * full + s)

    plsc.subcore_barrier()

    for w, (off, sz) in enumerate(ranges):
      @pl.when(s == w)
      def _():
        pltpu.sync_copy(dv_acc.at[pl.ds(off, sz)], dv_hbm.at[c].at[pl.ds(off, sz)])
        pltpu.sync_copy(de_acc.at[pl.ds(off, sz)], de_hbm.at[c].at[pl.ds(off, sz)])
    if extra is not None:
      off, sz = extra
      @pl.when(s == 0)
      def _():
        pltpu.sync_copy(dv_acc.at[pl.ds(off, sz)], dv_hbm.at[c].at[pl.ds(off, sz)])
        pltpu.sync_copy(de_acc.at[pl.ds(off, sz)], de_hbm.at[c].at[pl.ds(off, sz)])

  return deg


def _agg_kernel(num_inc, num_rows, d):
  mesh = plsc.VectorSubcoreMesh(core_axis_name="c", subcore_axis_name="s")
  n_chunks = num_inc // CHUNK
  per_core = n_chunks // NC
  full = per_core // NS
  tail_tiles = per_core % NS
  rows_per_tile = num_rows // NS

  @functools.partial(
      pl.kernel,
      out_type=jax.ShapeDtypeStruct((NC, num_rows, d), jnp.float32),
      mesh=mesh,
      scratch_types=[
          pltpu.VMEM((CHUNK,), jnp.int32),
          pltpu.VMEM((CHUNK,), jnp.int32),
          pltpu.VMEM((CHUNK, d), jnp.float32),
          pltpu.VMEM_SHARED((num_rows, d), jnp.float32),
      ],
  )
  def agg(table_hbm, gidx_hbm, sidx_hbm, z2_hbm, out_hbm, gi, si, rows, acc):
    c = lax.axis_index("c")
    s = lax.axis_index("s")

    for w in range(NS):
      @pl.when(s == w)
      def _():
        sl = pl.ds(w * rows_per_tile, rows_per_tile)
        pltpu.sync_copy(z2_hbm.at[sl], acc.at[sl])
    plsc.subcore_barrier()

    def do_chunk(chunk_id):
      base = chunk_id * CHUNK
      pltpu.sync_copy(gidx_hbm.at[pl.ds(base, CHUNK)], gi)
      pltpu.sync_copy(sidx_hbm.at[pl.ds(base, CHUNK)], si)
      pltpu.sync_copy(table_hbm.at[gi], rows)
      pltpu.sync_copy(rows, acc.at[si], add=True)

    @pl.loop(0, full)
    def _(j):
      do_chunk(c * per_core + s + NS * j)

    if tail_tiles:
      @pl.when(s < tail_tiles)
      def _():
        do_chunk(c * per_core + NS * full + s)

    plsc.subcore_barrier()

    for w in range(NS):
      @pl.when(s == w)
      def _():
        sl = pl.ds(w * rows_per_tile, rows_per_tile)
        pltpu.sync_copy(acc.at[sl], out_hbm.at[c].at[sl])

  return agg


def _scale1_call(x, dvp, dep, w_col):
  n = x.shape[0]

  def body(x_ref, dv_ref, de_ref, w_ref, xp_ref, sv_ref, se_ref):
    dv = jnp.maximum(dv_ref[0] + dv_ref[1], 1.0)
    de = jnp.maximum(de_ref[0] + de_ref[1], 1.0)
    sv = lax.rsqrt(dv)
    sv_ref[...] = sv
    se_ref[...] = w_ref[...] / de
    xp_ref[...] = x_ref[...] * sv

  return pl.pallas_call(
      body,
      out_shape=[
          jax.ShapeDtypeStruct((n, x.shape[1]), jnp.float32),
          jax.ShapeDtypeStruct((n, 1), jnp.float32),
          jax.ShapeDtypeStruct((n, 1), jnp.float32),
      ],
  )(x, dvp, dep, w_col)


def _scale2_call(hep, se_col):
  _, n, d = hep.shape

  def body(hep_ref, se_ref, out_ref):
    out_ref[...] = (hep_ref[0] + hep_ref[1]) * se_ref[...]

  return pl.pallas_call(
      body,
      out_shape=jax.ShapeDtypeStruct((n, d), jnp.float32),
  )(hep, se_col)


def _final_call(op, sv_col, W, b_row):
  _, n, d = op.shape

  def body(op_ref, sv_ref, w_ref, b_ref, out_ref):
    acc = (op_ref[0] + op_ref[1]) * sv_ref[...]
    out_ref[...] = lax.dot_general(
        acc, w_ref[...], (((1,), (1,)), ((), ())),
        preferred_element_type=jnp.float32) + b_ref[...]

  return pl.pallas_call(
      body,
      out_shape=jax.ShapeDtypeStruct((n, W.shape[0]), jnp.float32),
  )(op, sv_col, W, b_row)


@jax.jit
def kernel(x, hyperedge_index, hyperedge_weight, W, b):
  num_nodes, d_in = x.shape
  num_he = hyperedge_weight.shape[0]
  num_inc = hyperedge_index.shape[1]

  node_idx = hyperedge_index[0]
  he_idx = hyperedge_index[1]
  z1 = jnp.zeros((max(num_nodes, num_he),), jnp.float32)
  z2 = jnp.zeros((max(num_nodes, num_he), d_in), jnp.float32)

  dvp, dep = _deg_kernel(num_inc, num_he, num_nodes)(
      node_idx, he_idx, hyperedge_weight, z1)

  xp, sv_col, se_col = _scale1_call(
      x, dvp.reshape(NC, num_nodes, 1), dep.reshape(NC, num_he, 1),
      hyperedge_weight.reshape(num_he, 1))

  hep = _agg_kernel(num_inc, num_he, d_in)(xp, node_idx, he_idx, z2)
  hef = _scale2_call(hep, se_col)
  op = _agg_kernel(num_inc, num_nodes, d_in)(hef, he_idx, node_idx, z2)
  return _final_call(op, sv_col, W, b.reshape(1, -1))


# speedup vs baseline: 7.2475x; 7.2475x over previous
import functools

import jax
import jax.numpy as jnp
from jax import lax
from jax.experimental import pallas as pl
from jax.experimental.pallas import tpu as pltpu
from jax.experimental.pallas import tpu_sc as plsc

NC = 2
NS = 16
CHUNK = 128


def _tile_ranges(n, unit):
  chunks = n // unit
  base, rem = divmod(chunks, NS)
  out = []
  for w in range(NS):
    start = w * base + min(w, rem)
    cnt = base + (1 if w < rem else 0)
    out.append((start * unit, cnt * unit))
  return out


def _deg_kernel(num_inc, nhp, nvp):
  mesh = plsc.VectorSubcoreMesh(core_axis_name="c", subcore_axis_name="s")
  n_chunks = num_inc // CHUNK
  per_core = n_chunks // NC
  full = per_core // NS
  tail_tiles = per_core % NS

  @functools.partial(
      pl.kernel,
      out_type=[
          jax.ShapeDtypeStruct((NC, nvp), jnp.float32),
          jax.ShapeDtypeStruct((NC, nhp), jnp.float32),
      ],
      mesh=mesh,
      scratch_types=[
          pltpu.VMEM((CHUNK,), jnp.int32),
          pltpu.VMEM((CHUNK,), jnp.int32),
          pltpu.VMEM((CHUNK,), jnp.float32),
          pltpu.VMEM((CHUNK,), jnp.float32),
          pltpu.VMEM_SHARED((nvp,), jnp.float32),
          pltpu.VMEM_SHARED((nhp,), jnp.float32),
      ],
  )
  def deg(nidx_hbm, hidx_hbm, w_hbm, z1_hbm, dv_hbm, de_hbm,
          ni, hi, wv, ones, dv_acc, de_acc):
    c = lax.axis_index("c")
    s = lax.axis_index("s")

    @pl.loop(0, CHUNK, step=16)
    def _(i):
      ones[pl.ds(i, 16)] = jnp.full((16,), 1.0, jnp.float32)

    v_ranges = _tile_ranges(nvp, 128)
    e_ranges = _tile_ranges(nhp, 128)
    for w in range(NS):
      @pl.when(s == w)
      def _():
        off, sz = v_ranges[w]
        if sz:
          pltpu.sync_copy(z1_hbm.at[pl.ds(off, sz)], dv_acc.at[pl.ds(off, sz)])
        off, sz = e_ranges[w]
        if sz:
          pltpu.sync_copy(z1_hbm.at[pl.ds(off, sz)], de_acc.at[pl.ds(off, sz)])
    plsc.subcore_barrier()

    def do_chunk(chunk_id):
      base = chunk_id * CHUNK
      pltpu.sync_copy(nidx_hbm.at[pl.ds(base, CHUNK)], ni)
      pltpu.sync_copy(hidx_hbm.at[pl.ds(base, CHUNK)], hi)
      pltpu.sync_copy(w_hbm.at[hi], wv)
      pltpu.sync_copy(ones, de_acc.at[hi], add=True)
      pltpu.sync_copy(wv, dv_acc.at[ni], add=True)

    @pl.loop(0, full)
    def _(j):
      do_chunk(c * per_core + s + NS * j)

    if tail_tiles:
      @pl.when(s < tail_tiles)
      def _():
        do_chunk(c * per_core + NS * full + s)

    plsc.subcore_barrier()

    for w in range(NS):
      @pl.when(s == w)
      def _():
        off, sz = v_ranges[w]
        if sz:
          pltpu.sync_copy(dv_acc.at[pl.ds(off, sz)],
                          dv_hbm.at[c].at[pl.ds(off, sz)])
        off, sz = e_ranges[w]
        if sz:
          pltpu.sync_copy(de_acc.at[pl.ds(off, sz)],
                          de_hbm.at[c].at[pl.ds(off, sz)])

  return deg


def _agg_kernel(num_inc, num_rows, d):
  mesh = plsc.VectorSubcoreMesh(core_axis_name="c", subcore_axis_name="s")
  n_chunks = num_inc // CHUNK
  per_core = n_chunks // NC
  full = per_core // NS
  tail_tiles = per_core % NS
  row_ranges = _tile_ranges(num_rows, 8)

  @functools.partial(
      pl.kernel,
      out_type=jax.ShapeDtypeStruct((NC, num_rows, d), jnp.float32),
      mesh=mesh,
      scratch_types=[
          pltpu.VMEM((CHUNK,), jnp.int32),
          pltpu.VMEM((CHUNK,), jnp.int32),
          pltpu.VMEM((CHUNK, d), jnp.float32),
          pltpu.VMEM_SHARED((num_rows, d), jnp.float32),
      ],
  )
  def agg(table_hbm, gidx_hbm, sidx_hbm, z2_hbm, out_hbm, gi, si, rows, acc):
    c = lax.axis_index("c")
    s = lax.axis_index("s")

    for w in range(NS):
      @pl.when(s == w)
      def _():
        off, sz = row_ranges[w]
        if sz:
          sl = pl.ds(off, sz)
          pltpu.sync_copy(z2_hbm.at[sl], acc.at[sl])
    plsc.subcore_barrier()

    def do_chunk(chunk_id):
      base = chunk_id * CHUNK
      pltpu.sync_copy(gidx_hbm.at[pl.ds(base, CHUNK)], gi)
      pltpu.sync_copy(sidx_hbm.at[pl.ds(base, CHUNK)], si)
      pltpu.sync_copy(table_hbm.at[gi], rows)
      pltpu.sync_copy(rows, acc.at[si], add=True)

    @pl.loop(0, full)
    def _(j):
      do_chunk(c * per_core + s + NS * j)

    if tail_tiles:
      @pl.when(s < tail_tiles)
      def _():
        do_chunk(c * per_core + NS * full + s)

    plsc.subcore_barrier()

    for w in range(NS):
      @pl.when(s == w)
      def _():
        off, sz = row_ranges[w]
        if sz:
          sl = pl.ds(off, sz)
          pltpu.sync_copy(acc.at[sl], out_hbm.at[c].at[sl])

  return agg


def _scale1_call(x, dvp, dep, w_col):
  n = x.shape[0]

  def body(x_ref, dv_ref, de_ref, w_ref, xp_ref, sv_ref, se_ref):
    dv = jnp.maximum(dv_ref[0] + dv_ref[1], 1.0)
    de = jnp.maximum(de_ref[0] + de_ref[1], 1.0)
    sv = lax.rsqrt(dv)
    sv_ref[...] = sv
    se_ref[...] = w_ref[...] / de
    xp_ref[...] = x_ref[...] * sv

  return pl.pallas_call(
      body,
      out_shape=[
          jax.ShapeDtypeStruct((n, x.shape[1]), jnp.float32),
          jax.ShapeDtypeStruct((n, 1), jnp.float32),
          jax.ShapeDtypeStruct((n, 1), jnp.float32),
      ],
  )(x, dvp, dep, w_col)


def _scale2_call(hep, se_col):
  _, n, d = hep.shape

  def body(hep_ref, se_ref, out_ref):
    out_ref[...] = (hep_ref[0] + hep_ref[1]) * se_ref[...]

  return pl.pallas_call(
      body,
      out_shape=jax.ShapeDtypeStruct((n, d), jnp.float32),
  )(hep, se_col)


def _final_call(op, sv_col, W, b_row):
  _, n, d = op.shape

  def body(op_ref, sv_ref, w_ref, b_ref, out_ref):
    acc = (op_ref[0] + op_ref[1]) * sv_ref[...]
    out_ref[...] = lax.dot_general(
        acc, w_ref[...], (((1,), (1,)), ((), ())),
        precision=lax.Precision.HIGHEST,
        preferred_element_type=jnp.float32) + b_ref[...]

  return pl.pallas_call(
      body,
      out_shape=jax.ShapeDtypeStruct((n, W.shape[0]), jnp.float32),
  )(op, sv_col, W, b_row)


@jax.jit
def kernel(x, hyperedge_index, hyperedge_weight, W, b):
  num_nodes, d_in = x.shape
  num_he = hyperedge_weight.shape[0]
  num_inc = hyperedge_index.shape[1]

  node_idx = hyperedge_index[0]
  he_idx = hyperedge_index[1]
  nvp = -(-num_nodes // 128) * 128
  nhp = -(-num_he // 128) * 128
  z1 = jnp.zeros((max(nvp, nhp),), jnp.float32)
  z2 = jnp.zeros((max(num_nodes, num_he), d_in), jnp.float32)

  dvp, dep = _deg_kernel(num_inc, nhp, nvp)(
      node_idx, he_idx, hyperedge_weight, z1)

  xp, sv_col, se_col = _scale1_call(
      x, dvp[:, :num_nodes].reshape(NC, num_nodes, 1),
      dep[:, :num_he].reshape(NC, num_he, 1),
      hyperedge_weight.reshape(num_he, 1))

  hep = _agg_kernel(num_inc, num_he, d_in)(xp, node_idx, he_idx, z2)
  hef = _scale2_call(hep, se_col)
  op = _agg_kernel(num_inc, num_nodes, d_in)(hef, he_idx, node_idx, z2)
  return _final_call(op, sv_col, W, b.reshape(1, -1))
